# Initial kernel scaffold; baseline (speedup 1.0000x reference)
import jax
import jax.numpy as jnp
from jax.experimental import pallas as pl


def kernel(x, edge_index, params):
    raise NotImplementedError("write your pallas kernel here")


# speedup vs baseline: 1.3961x; 1.3961x over previous
import functools

import jax
import jax.numpy as jnp
import numpy as np
from jax.experimental import pallas as pl

_DROP = np.int32(2**30)


def _pick_block(d, prefs):
    for c in prefs:
        if d >= c and d % c == 0:
            return c
    return d


def _mm(a, b):
    M, K = a.shape
    K2, N = b.shape
    assert K == K2
    bm = _pick_block(M, (512, 256, 128))
    bn = _pick_block(N, (512, 256, 128))
    bk = _pick_block(K, (2048, 1024, 512, 256, 128))

    def kern(a_ref, b_ref, o_ref):
        @pl.when(pl.program_id(2) == 0)
        def _():
            o_ref[...] = jnp.zeros_like(o_ref)

        o_ref[...] += jnp.dot(a_ref[...], b_ref[...],
                              preferred_element_type=jnp.float32)

    return pl.pallas_call(
        kern,
        grid=(M // bm, N // bn, K // bk),
        in_specs=[
            pl.BlockSpec((bm, bk), lambda i, j, k: (i, k)),
            pl.BlockSpec((bk, bn), lambda i, j, k: (k, j)),
        ],
        out_specs=pl.BlockSpec((bm, bn), lambda i, j, k: (i, j)),
        out_shape=jax.ShapeDtypeStruct((M, N), jnp.float32),
    )(a, b)


def _mmT(a, b):
    K, M = a.shape
    K2, N = b.shape
    assert K == K2
    bm = _pick_block(M, (512, 256, 128))
    bn = _pick_block(N, (512, 256, 128))
    bk = _pick_block(K, (1024, 512, 256, 128))

    def kern(a_ref, b_ref, o_ref):
        @pl.when(pl.program_id(2) == 0)
        def _():
            o_ref[...] = jnp.zeros_like(o_ref)

        o_ref[...] += jax.lax.dot_general(
            a_ref[...], b_ref[...], (((0,), (0,)), ((), ())),
            preferred_element_type=jnp.float32)

    return pl.pallas_call(
        kern,
        grid=(M // bm, N // bn, K // bk),
        in_specs=[
            pl.BlockSpec((bk, bm), lambda i, j, k: (k, i)),
            pl.BlockSpec((bk, bn), lambda i, j, k: (k, j)),
        ],
        out_specs=pl.BlockSpec((bm, bn), lambda i, j, k: (i, j)),
        out_shape=jax.ShapeDtypeStruct((M, N), jnp.float32),
    )(a, b)


def _pad_rows(m, rows):
    return jnp.pad(m, ((0, rows - m.shape[0]), (0, 0)))


def _gcn_inner(A, x, W, b, n):
    P = A.shape[0]
    deg = jnp.sum(A, axis=0) + 2.0
    dis = deg ** -0.5
    z = dis[:, None] * _mm(x, W)
    out = dis[:, None] * (_mmT(A, z) + 2.0 * z) + b[None, :]
    mask = (jnp.arange(P) < n)[:, None]
    return jnp.where(mask, out, 0.0)


def _pool_stats(x, p, n, k):
    score = jnp.tanh((x[:n] @ p) / jnp.linalg.norm(p))
    vals, perm = jax.lax.top_k(score, k)
    return vals, perm


def _augment_pool(A, perm, k_pad):
    P = A.shape[0]
    k = perm.shape[0]
    ar = jnp.arange(k, dtype=jnp.int32)
    R = jnp.zeros((k_pad, P), jnp.float32)
    R = R.at[:k, :].set(A[perm, :])
    R = R.at[ar, perm].set(1.0)
    Cc = jnp.zeros((P, k_pad), jnp.float32)
    Cc = Cc.at[:, :k].set(A[:, perm])
    Cc = Cc.at[perm, ar].set(1.0)
    Ap = _mm(R, Cc)
    d = jnp.arange(k_pad)
    return Ap.at[d, d].set(0.0)


def kernel(x, edge_index, params):
    N, C = x.shape
    depth = len(params['pw'])
    ratio = 0.5

    def pad_to(v, m=512):
        return int(-(-v // m) * m)

    n0 = N
    k1 = int(np.ceil(ratio * n0))
    k2 = int(np.ceil(ratio * k1))
    k3 = int(np.ceil(ratio * k2))
    P0, P1, P2, P3 = pad_to(n0), pad_to(k1), pad_to(k2), pad_to(k3)

    src = edge_index[0]
    dst = edge_index[1]
    is_self = src == dst
    dst_ns = jnp.where(is_self, _DROP, dst)
    self_dst = jnp.where(is_self, dst, _DROP)

    ones_e = jnp.ones_like(src, jnp.float32)
    indeg_ns = jnp.zeros((n0,), jnp.float32).at[dst_ns].add(ones_e)
    self_cnt = jnp.zeros((n0,), jnp.float32).at[self_dst].add(ones_e)
    diagval = jnp.where(self_cnt > 0, self_cnt, 2.0)
    deg0 = indeg_ns + diagval
    dis0 = deg0 ** -0.5
    dis0P = _pad_rows(dis0[:, None], P0)[:, 0]
    diagvalP = _pad_rows(diagval[:, None], P0)[:, 0]
    mask0 = (jnp.arange(P0) < n0)[:, None]

    def gcn0(xp, W, b):
        z = dis0P[:, None] * _mm(xp, W)
        agg = jnp.zeros((P0, C), jnp.float32).at[dst_ns].add(z[src])
        out = dis0P[:, None] * (agg + diagvalP[:, None] * z) + b[None, :]
        return jnp.where(mask0, out, 0.0)

    xP = _pad_rows(x, P0)

    x1 = jax.nn.relu(gcn0(xP, params['Wd'][0], params['bd'][0]))

    vals1, perm1 = _pool_stats(x1, params['pw'][0], n0, k1)
    inv1 = jnp.full((n0,), _DROP, jnp.int32).at[perm1].set(
        jnp.arange(k1, dtype=jnp.int32))
    src_kept = jnp.where(is_self, _DROP, inv1[src])
    dst_kept = jnp.where(is_self, _DROP, inv1[dst])
    ar1 = jnp.arange(k1, dtype=jnp.int32)
    R = jnp.zeros((P1, P0), jnp.float32).at[src_kept, dst].add(ones_e)
    R = R.at[ar1, perm1].set(1.0)
    Cc = jnp.zeros((P0, P1), jnp.float32).at[src, dst_kept].add(ones_e)
    Cc = Cc.at[perm1, ar1].set(1.0)
    A1 = _mm(R, Cc)
    d1 = jnp.arange(P1)
    A1 = A1.at[d1, d1].set(0.0)

    xp1 = _pad_rows(x1[perm1] * vals1[:, None], P1)
    x2 = jax.nn.relu(_gcn_inner(A1, xp1, params['Wd'][1], params['bd'][1], k1))

    vals2, perm2 = _pool_stats(x2, params['pw'][1], k1, k2)
    A2 = _augment_pool(A1, perm2, P2)
    xp2 = _pad_rows(x2[perm2] * vals2[:, None], P2)
    x3 = jax.nn.relu(_gcn_inner(A2, xp2, params['Wd'][2], params['bd'][2], k2))

    vals3, perm3 = _pool_stats(x3, params['pw'][2], k2, k3)
    A3 = _augment_pool(A2, perm3, P3)
    xp3 = _pad_rows(x3[perm3] * vals3[:, None], P3)
    x4 = jax.nn.relu(_gcn_inner(A3, xp3, params['Wd'][3], params['bd'][3], k3))

    u2 = x3 + jnp.zeros((P2, C), jnp.float32).at[perm3].set(x4[:k3])
    y2 = jax.nn.relu(_gcn_inner(A2, u2, params['Wu'][0], params['bu'][0], k2))

    u1 = x2 + jnp.zeros((P1, C), jnp.float32).at[perm2].set(y2[:k2])
    y1 = jax.nn.relu(_gcn_inner(A1, u1, params['Wu'][1], params['bu'][1], k1))

    u0 = x1 + jnp.zeros((P0, C), jnp.float32).at[perm1].set(y1[:k1])
    y0 = gcn0(u0, params['Wu'][2], params['bu'][2])

    return jax.nn.relu(y0[:N])
